# baseline (speedup 1.0000x reference)
import functools
import math

import jax
import jax.numpy as jnp
from jax import lax
from jax.experimental import pallas as pl
from jax.experimental.pallas import tpu as pltpu
from jax.experimental.pallas import tpu_sc as plsc

NC = 2
NS = 16
NW = NC * NS

CHUNK = 32


def _sc_gather(table, idx):
    B = idx.shape[0]
    V, D = table.shape
    b_per_w = B // NW
    n_chunks = b_per_w // CHUNK
    mesh = plsc.VectorSubcoreMesh(core_axis_name="c", subcore_axis_name="s")

    @functools.partial(
        pl.kernel,
        mesh=mesh,
        out_type=jax.ShapeDtypeStruct((B, D), jnp.float32),
        scratch_types=[
            pltpu.VMEM((b_per_w,), jnp.int32),
            pltpu.VMEM((CHUNK, D), jnp.float32),
            pltpu.SemaphoreType.DMA,
        ],
    )
    def k(table_hbm, idx_hbm, out_hbm, idx_v, rows_v, sem):
        wid = lax.axis_index("s") * NC + lax.axis_index("c")
        base = wid * b_per_w
        pltpu.sync_copy(idx_hbm.at[pl.ds(base, b_per_w)], idx_v)

        @pl.loop(0, n_chunks)
        def _(c):
            off = c * CHUNK
            pltpu.async_copy(
                table_hbm.at[idx_v.at[pl.ds(off, CHUNK)]], rows_v, sem
            ).wait()
            pltpu.sync_copy(rows_v, out_hbm.at[pl.ds(base + off, CHUNK)])

    return k(table, idx)


def _tc_scale_add(g, pe, scale):
    S, B, D = g.shape
    BS = 256

    def body(g_ref, pe_ref, o_ref):
        o_ref[...] = g_ref[...] * scale + pe_ref[...]

    return pl.pallas_call(
        body,
        grid=(S // BS,),
        in_specs=[
            pl.BlockSpec((BS, B, D), lambda i: (i, 0, 0)),
            pl.BlockSpec((BS, 1, D), lambda i: (i, 0, 0)),
        ],
        out_specs=pl.BlockSpec((BS, B, D), lambda i: (i, 0, 0)),
        out_shape=jax.ShapeDtypeStruct((S, B, D), jnp.float32),
    )(g, pe)


def kernel(x, emb_table, pe):
    S, B = x.shape
    V, D = emb_table.shape
    idx = x.reshape(-1).astype(jnp.int32)
    g = _sc_gather(emb_table, idx).reshape(S, B, D)
    return _tc_scale_add(g, pe[:S], math.sqrt(D))


# speedup vs baseline: 1.3358x; 1.3358x over previous
import functools
import math

import jax
import jax.numpy as jnp
from jax import lax
from jax.experimental import pallas as pl
from jax.experimental.pallas import tpu as pltpu
from jax.experimental.pallas import tpu_sc as plsc

NC = 2
NS = 16
NW = NC * NS

CHUNK = 32


def _sc_gather(table, idx):
    B = idx.shape[0]
    V, D = table.shape
    b_per_w = B // NW
    n_chunks = b_per_w // CHUNK
    mesh = plsc.VectorSubcoreMesh(core_axis_name="c", subcore_axis_name="s")

    @functools.partial(
        pl.kernel,
        mesh=mesh,
        out_type=jax.ShapeDtypeStruct((B, D), jnp.float32),
        scratch_types=[
            pltpu.VMEM((b_per_w,), jnp.int32),
            pltpu.VMEM((CHUNK, D), jnp.float32),
            pltpu.SemaphoreType.DMA,
        ],
    )
    def k(table_hbm, idx_hbm, out_hbm, idx_v, rows_v, sem):
        wid = lax.axis_index("s") * NC + lax.axis_index("c")
        base = wid * b_per_w
        pltpu.sync_copy(idx_hbm.at[pl.ds(base, b_per_w)], idx_v)

        @pl.loop(0, n_chunks)
        def _(c):
            off = c * CHUNK
            pltpu.async_copy(
                table_hbm.at[idx_v.at[pl.ds(off, CHUNK)]], rows_v, sem
            ).wait()
            pltpu.sync_copy(rows_v, out_hbm.at[pl.ds(base + off, CHUNK)])

    return k(table, idx)


def _tc_scale_add(g, pe2d, scale, batch):
    SB, D = g.shape
    S = SB // batch
    BS = 256

    def body(g_ref, pe_ref, o_ref):
        g3 = g_ref[...].reshape(BS, batch, D)
        o_ref[...] = g3 * scale + pe_ref[...][:, None, :]

    return pl.pallas_call(
        body,
        grid=(S // BS,),
        in_specs=[
            pl.BlockSpec((BS * batch, D), lambda i: (i, 0)),
            pl.BlockSpec((BS, D), lambda i: (i, 0)),
        ],
        out_specs=pl.BlockSpec((BS, batch, D), lambda i: (i, 0, 0)),
        out_shape=jax.ShapeDtypeStruct((S, batch, D), jnp.float32),
    )(g, pe2d)


def kernel(x, emb_table, pe):
    S, B = x.shape
    V, D = emb_table.shape
    idx = x.reshape(-1).astype(jnp.int32)
    g = _sc_gather(emb_table, idx)
    pe2d = pe[:S].reshape(S, D)
    return _tc_scale_add(g, pe2d, math.sqrt(D), B)


# speedup vs baseline: 1.3850x; 1.0368x over previous
import functools
import math

import jax
import jax.numpy as jnp
from jax import lax
from jax.experimental import pallas as pl
from jax.experimental.pallas import tpu as pltpu
from jax.experimental.pallas import tpu_sc as plsc

NC = 2
NS = 16
NW = NC * NS

CHUNK = 32


def _sc_gather(table, idx):
    B = idx.shape[0]
    V, D = table.shape
    b_per_w = B // NW
    n_chunks = b_per_w // CHUNK
    mesh = plsc.VectorSubcoreMesh(core_axis_name="c", subcore_axis_name="s")

    @functools.partial(
        pl.kernel,
        mesh=mesh,
        out_type=jax.ShapeDtypeStruct((B, D), jnp.float32),
        scratch_types=[
            pltpu.VMEM((b_per_w,), jnp.int32),
            pltpu.VMEM((CHUNK, D), jnp.float32),
            pltpu.VMEM((CHUNK, D), jnp.float32),
            pltpu.SemaphoreType.DMA,
            pltpu.SemaphoreType.DMA,
            pltpu.SemaphoreType.DMA,
            pltpu.SemaphoreType.DMA,
        ],
    )
    def k(table_hbm, idx_hbm, out_hbm, idx_v, buf0, buf1, g0, g1, w0, w1):
        wid = lax.axis_index("s") * NC + lax.axis_index("c")
        base = wid * b_per_w
        pltpu.sync_copy(idx_hbm.at[pl.ds(base, b_per_w)], idx_v)

        def g_copy(c, buf, sem):
            return pltpu.make_async_copy(
                table_hbm.at[idx_v.at[pl.ds(c * CHUNK, CHUNK)]], buf, sem
            )

        def w_copy(c, buf, sem):
            return pltpu.make_async_copy(
                buf, out_hbm.at[pl.ds(base + c * CHUNK, CHUNK)], sem
            )

        g_copy(0, buf0, g0).start()
        g_copy(1, buf1, g1).start()

        @pl.loop(0, n_chunks - 2, step=2)
        def _(c):
            g_copy(c, buf0, g0).wait()
            w_copy(c, buf0, w0).start()
            g_copy(c + 1, buf1, g1).wait()
            w_copy(c + 1, buf1, w1).start()
            w_copy(c, buf0, w0).wait()
            g_copy(c + 2, buf0, g0).start()
            w_copy(c + 1, buf1, w1).wait()
            g_copy(c + 3, buf1, g1).start()

        g_copy(n_chunks - 2, buf0, g0).wait()
        w_copy(n_chunks - 2, buf0, w0).start()
        g_copy(n_chunks - 1, buf1, g1).wait()
        w_copy(n_chunks - 1, buf1, w1).start()
        w_copy(n_chunks - 2, buf0, w0).wait()
        w_copy(n_chunks - 1, buf1, w1).wait()

    return k(table, idx)


def _tc_scale_add(g, pe2d, scale, batch):
    SB, D = g.shape
    S = SB // batch
    BS = 256

    def body(g_ref, pe_ref, o_ref):
        g3 = g_ref[...].reshape(BS, batch, D)
        o_ref[...] = g3 * scale + pe_ref[...][:, None, :]

    return pl.pallas_call(
        body,
        grid=(S // BS,),
        in_specs=[
            pl.BlockSpec((BS * batch, D), lambda i: (i, 0)),
            pl.BlockSpec((BS, D), lambda i: (i, 0)),
        ],
        out_specs=pl.BlockSpec((BS, batch, D), lambda i: (i, 0, 0)),
        out_shape=jax.ShapeDtypeStruct((S, batch, D), jnp.float32),
    )(g, pe2d)


def kernel(x, emb_table, pe):
    S, B = x.shape
    V, D = emb_table.shape
    idx = x.reshape(-1).astype(jnp.int32)
    g = _sc_gather(emb_table, idx)
    pe2d = pe[:S].reshape(S, D)
    return _tc_scale_add(g, pe2d, math.sqrt(D), B)


# speedup vs baseline: 1.7226x; 1.2438x over previous
import functools
import math

import jax
import jax.numpy as jnp
from jax import lax
from jax.experimental import pallas as pl
from jax.experimental.pallas import tpu as pltpu
from jax.experimental.pallas import tpu_sc as plsc

NC = 2
NS = 16
NW = NC * NS

CHUNK = 32


def _sc_gather(table, idx):
    B = idx.shape[0]
    V, D = table.shape
    b_per_w = B // NW
    n_chunks = b_per_w // CHUNK
    mesh = plsc.VectorSubcoreMesh(core_axis_name="c", subcore_axis_name="s")

    @functools.partial(
        pl.kernel,
        mesh=mesh,
        out_type=jax.ShapeDtypeStruct((B, D), jnp.float32),
        scratch_types=[
            pltpu.VMEM((b_per_w,), jnp.int32),
            pltpu.VMEM((CHUNK, D), jnp.float32),
            pltpu.VMEM((CHUNK, D), jnp.float32),
            pltpu.SemaphoreType.DMA,
            pltpu.SemaphoreType.DMA,
            pltpu.SemaphoreType.DMA,
            pltpu.SemaphoreType.DMA,
        ],
    )
    def k(table_hbm, idx_hbm, out_hbm, idx_v, buf0, buf1, g0, g1, w0, w1):
        wid = lax.axis_index("s") * NC + lax.axis_index("c")
        base = wid * b_per_w
        pltpu.sync_copy(idx_hbm.at[pl.ds(base, b_per_w)], idx_v)

        def g_copy(c, buf, sem):
            return pltpu.make_async_copy(
                table_hbm.at[idx_v.at[pl.ds(c * CHUNK, CHUNK)]], buf, sem
            )

        def w_copy(c, buf, sem):
            return pltpu.make_async_copy(
                buf, out_hbm.at[pl.ds(base + c * CHUNK, CHUNK)], sem
            )

        g_copy(0, buf0, g0).start()
        g_copy(1, buf1, g1).start()

        @pl.loop(0, n_chunks - 2, step=2)
        def _(c):
            g_copy(c, buf0, g0).wait()
            w_copy(c, buf0, w0).start()
            g_copy(c + 1, buf1, g1).wait()
            w_copy(c + 1, buf1, w1).start()
            w_copy(c, buf0, w0).wait()
            g_copy(c + 2, buf0, g0).start()
            w_copy(c + 1, buf1, w1).wait()
            g_copy(c + 3, buf1, g1).start()

        g_copy(n_chunks - 2, buf0, g0).wait()
        w_copy(n_chunks - 2, buf0, w0).start()
        g_copy(n_chunks - 1, buf1, g1).wait()
        w_copy(n_chunks - 1, buf1, w1).start()
        w_copy(n_chunks - 2, buf0, w0).wait()
        w_copy(n_chunks - 1, buf1, w1).wait()

    return k(table, idx)


def _tc_scale_add(g, pe, S, scale, batch):
    SB, D = g.shape
    BS = 512

    def body(g_ref, pe_ref, o_ref):
        g3 = g_ref[...].reshape(BS, batch, D)
        o_ref[...] = g3 * scale + pe_ref[...]

    return pl.pallas_call(
        body,
        grid=(S // BS,),
        in_specs=[
            pl.BlockSpec((BS * batch, D), lambda i: (i, 0)),
            pl.BlockSpec((BS, 1, D), lambda i: (i, 0, 0)),
        ],
        out_specs=pl.BlockSpec((BS, batch, D), lambda i: (i, 0, 0)),
        out_shape=jax.ShapeDtypeStruct((S, batch, D), jnp.float32),
    )(g, pe)


def kernel(x, emb_table, pe):
    S, B = x.shape
    V, D = emb_table.shape
    idx = x.reshape(-1).astype(jnp.int32)
    g = _sc_gather(emb_table, idx)
    return _tc_scale_add(g, pe, S, math.sqrt(D), B)
